# Initial kernel scaffold; baseline (speedup 1.0000x reference)
import jax
import jax.numpy as jnp
from jax.experimental import pallas as pl


def kernel(x, edge_index, W0, b0, gamma0, beta0, W1, b1, gamma1, beta1, W2, b2):
    raise NotImplementedError("write your pallas kernel here")


# speedup vs baseline: 3.1582x; 3.1582x over previous
import functools

import jax
import jax.numpy as jnp
from jax import lax
from jax.experimental import pallas as pl
from jax.experimental.pallas import tpu as pltpu
from jax.experimental.pallas import tpu_sc as plsc

N = 10000
NP = 10240
E = 320000
CH = 128
NCH = 2560
EP = NCH * CH
NC, NS = 2, 16
NW = NC * NS
CPW = NCH // NW
RPT = NP // NS
DUMMY = N
D_H = 128
D_OUT = 64
N_CLS = 40
BN_EPS = 1e-5

_mesh = plsc.VectorSubcoreMesh(core_axis_name="c", subcore_axis_name="s")
_f32 = jnp.float32


def _deg_body(srcs, dsts, out, idx_s, idx_d, ones, zbuf, acc_o, acc_i):
  cid = lax.axis_index("c")
  sid = lax.axis_index("s")
  wid = cid * NS + sid

  for k in range(CH // 16):
    ones[pl.ds(k * 16, 16)] = jnp.ones((16,), _f32)

  def zb(i, _):
    zbuf[pl.ds(i * 16, 16)] = jnp.zeros((16,), _f32)
    return 0
  lax.fori_loop(0, RPT // 16, zb, 0)
  pltpu.sync_copy(zbuf, acc_o.at[pl.ds(sid * RPT, RPT)])
  pltpu.sync_copy(zbuf, acc_i.at[pl.ds(sid * RPT, RPT)])
  plsc.subcore_barrier()

  pltpu.sync_copy(srcs.at[pl.ds(wid * CPW, CPW), :], idx_s)
  pltpu.sync_copy(dsts.at[pl.ds(wid * CPW, CPW), :], idx_d)

  def body(j, _):
    pltpu.sync_copy(ones, acc_o.at[idx_s.at[j]], add=True)
    pltpu.sync_copy(ones, acc_i.at[idx_d.at[j]], add=True)
    return 0
  lax.fori_loop(0, CPW, body, 0)

  plsc.subcore_barrier()
  pltpu.sync_copy(acc_o.at[pl.ds(sid * RPT, RPT)],
                  out.at[cid, 0, pl.ds(sid * RPT, RPT)])
  pltpu.sync_copy(acc_i.at[pl.ds(sid * RPT, RPT)],
                  out.at[cid, 1, pl.ds(sid * RPT, RPT)])


_deg_call = pl.kernel(
    _deg_body,
    out_type=jax.ShapeDtypeStruct((NC, 2, NP), _f32),
    mesh=_mesh,
    scratch_types=[
        pltpu.VMEM((CPW, CH), jnp.int32),
        pltpu.VMEM((CPW, CH), jnp.int32),
        pltpu.VMEM((CH,), _f32),
        pltpu.VMEM((RPT,), _f32),
        pltpu.VMEM_SHARED((NP,), _f32),
        pltpu.VMEM_SHARED((NP,), _f32),
    ],
)


def _make_agg(d):
  def body(h, srcs, dsts, out, idx_s, idx_d, rows, acc, sem):
    cid = lax.axis_index("c")
    sid = lax.axis_index("s")
    wid = cid * NS + sid

    def zb(r, _):
      for k in range(d // 16):
        rows[r, pl.ds(k * 16, 16)] = jnp.zeros((16,), _f32)
      return 0
    lax.fori_loop(0, CH, zb, 0)
    for t in range(RPT // CH):
      pltpu.sync_copy(rows, acc.at[pl.ds(sid * RPT + t * CH, CH), :])
    plsc.subcore_barrier()

    pltpu.sync_copy(srcs.at[pl.ds(wid * CPW, CPW), :], idx_s)
    pltpu.sync_copy(dsts.at[pl.ds(wid * CPW, CPW), :], idx_d)

    def step(j, _):
      pltpu.async_copy(h.at[idx_s.at[j]], rows, sem).wait()
      pltpu.sync_copy(rows, acc.at[idx_d.at[j]], add=True)
      return 0
    lax.fori_loop(0, CPW, step, 0)

    plsc.subcore_barrier()
    pltpu.sync_copy(acc.at[pl.ds(sid * RPT, RPT), :],
                    out.at[cid, pl.ds(sid * RPT, RPT), :])

  return pl.kernel(
      body,
      out_type=jax.ShapeDtypeStruct((NC, NP, d), _f32),
      mesh=_mesh,
      scratch_types=[
          pltpu.VMEM((CPW, CH), jnp.int32),
          pltpu.VMEM((CPW, CH), jnp.int32),
          pltpu.VMEM((CH, d), _f32),
          pltpu.VMEM_SHARED((NP, d), _f32),
          pltpu.SemaphoreType.DMA,
      ],
  )


_agg128 = _make_agg(D_H)


def _norm_body(deg_ref, out_ref):
  deg = deg_ref[0] + deg_ref[1]
  out_ref[...] = lax.rsqrt(jnp.clip(deg, 1.0, None))


def _norms(degp):
  return pl.pallas_call(
      _norm_body,
      out_shape=jax.ShapeDtypeStruct((2, NP), _f32),
  )(degp)


def _prep_body(x_ref, ns_ref, w_ref, out_ref):
  g = x_ref[...] * ns_ref[...]
  out_ref[...] = jnp.dot(g, w_ref[...], preferred_element_type=_f32)


def _prep(x, ns_col, w0):
  return pl.pallas_call(
      _prep_body,
      out_shape=jax.ShapeDtypeStruct((NP, D_H), _f32),
  )(x, ns_col, w0)


def _mid_body(w_ref, p_ref, nd_ref, b_ref, ga_ref, be_ref, ns_ref, out_ref):
  a = (p_ref[0] + p_ref[1]) * nd_ref[...] + b_ref[...]
  rmask = (lax.broadcasted_iota(jnp.int32, (NP, 1), 0) < N).astype(_f32)
  mean = jnp.sum(a * rmask, axis=0, keepdims=True) / N
  diff = (a - mean) * rmask
  var = jnp.sum(diff * diff, axis=0, keepdims=True) / N
  h = ga_ref[...] * (a - mean) * lax.rsqrt(var + BN_EPS) + be_ref[...]
  h = jnp.maximum(h, 0.0) * rmask * ns_ref[...]
  if w_ref is None:
    out_ref[...] = h
  else:
    out_ref[...] = jnp.dot(h, w_ref[...], preferred_element_type=_f32)


def _mid(p, nd_col, b, ga, be, ns_col, w=None):
  if w is None:
    return pl.pallas_call(
        functools.partial(_mid_body, None),
        out_shape=jax.ShapeDtypeStruct((NP, D_H), _f32),
    )(p, nd_col, b, ga, be, ns_col)
  return pl.pallas_call(
      _mid_body,
      out_shape=jax.ShapeDtypeStruct((NP, D_H), _f32),
  )(w, p, nd_col, b, ga, be, ns_col)


def _final_body(p_ref, nd_ref, w_ref, b_ref, out_ref):
  agg = (p_ref[0] + p_ref[1]) * nd_ref[...]
  s = jnp.dot(agg, w_ref[...], preferred_element_type=_f32) + b_ref[...]
  cmask = lax.broadcasted_iota(jnp.int32, (1, D_OUT), 1) < N_CLS
  sm = jnp.where(cmask, s, -jnp.inf)
  m = jnp.max(sm, axis=1, keepdims=True)
  e = jnp.where(cmask, jnp.exp(s - m), 0.0)
  lse = jnp.log(jnp.sum(e, axis=1, keepdims=True))
  out_ref[...] = s - m - lse


def _final(p, nd_col, w2p, b2p):
  return pl.pallas_call(
      _final_body,
      out_shape=jax.ShapeDtypeStruct((NP, D_OUT), _f32),
  )(p, nd_col, w2p, b2p)


def kernel(x, edge_index, W0, b0, gamma0, beta0, W1, b1, gamma1, beta1, W2, b2):
  pad = jnp.full((EP - E,), DUMMY, jnp.int32)
  srcs = jnp.concatenate([edge_index[0], pad]).reshape(NCH, CH)
  dsts = jnp.concatenate([edge_index[1], pad]).reshape(NCH, CH)
  x_pad = jnp.pad(x, ((0, NP - N), (0, 0)))
  w2p = jnp.pad(W2, ((0, 0), (0, D_OUT - N_CLS)))
  b2p = jnp.pad(b2, (0, D_OUT - N_CLS))

  degp = _deg_call(srcs, dsts)
  norms = _norms(degp)
  ns_col = norms[0][:, None]
  nd_col = norms[1][:, None]

  hw0 = _prep(x_pad, ns_col, W0)
  p0 = _agg128(hw0, srcs, dsts)
  hw1 = _mid(p0, nd_col, b0, gamma0, beta0, ns_col, W1)
  p1 = _agg128(hw1, srcs, dsts)
  g2 = _mid(p1, nd_col, b1, gamma1, beta1, ns_col)
  p2 = _agg128(g2, srcs, dsts)
  out = _final(p2, nd_col, w2p, b2p)
  return out[:N, :N_CLS]
